# Initial kernel scaffold; baseline (speedup 1.0000x reference)
import jax
import jax.numpy as jnp
from jax.experimental import pallas as pl


def kernel(sample, entity_embedding, relation_embedding):
    raise NotImplementedError("write your pallas kernel here")


# speedup vs baseline: 2.8023x; 2.8023x over previous
import functools

import jax
import jax.numpy as jnp
from jax import lax
from jax.experimental import pallas as pl
from jax.experimental.pallas import tpu as pltpu, tpu_sc as plsc

HD = 256
ED = 2 * HD
NW = 32
NCH = 16
CH = 32
L = 16


def _sc_body(hi_hbm, ri_hbm, ti_hbm, ent_hbm, rel_hbm, out_hbm,
             hi_v, ri_v, ti_v, hbuf, rbuf, tbuf, score_v, sem0, sem1):
    wid = lax.axis_index("s") * 2 + lax.axis_index("c")
    bw = NCH * CH

    pltpu.sync_copy(hi_hbm.at[wid], hi_v)
    pltpu.sync_copy(ri_hbm.at[wid], ri_v)
    pltpu.sync_copy(ti_hbm.at[wid], ti_v)

    sems = (sem0, sem1)

    def issue(c):
        slot = c & 1
        s = sems[slot]
        return (
            pltpu.async_copy(ent_hbm.at[hi_v.at[c]], hbuf.at[slot], s),
            pltpu.async_copy(rel_hbm.at[ri_v.at[c]], rbuf.at[slot], s),
            pltpu.async_copy(ent_hbm.at[ti_v.at[c]], tbuf.at[slot], s),
        )

    lane = lax.iota(jnp.int32, L)
    lane0 = lane == 0

    cps = [None, None]
    cps[0] = issue(0)
    for c in range(NCH):
        slot = c & 1
        if c + 1 < NCH:
            cps[(c + 1) & 1] = issue(c + 1)
        for cp in cps[slot]:
            cp.wait()

        def body(s, carry, _slot=slot, _c=c):
            acc = jnp.zeros((L,), jnp.float32)
            for j in range(HD // L):
                rh = hbuf[_slot, s, pl.ds(j * L, L)]
                ih = hbuf[_slot, s, pl.ds(HD + j * L, L)]
                rr = rbuf[_slot, s, pl.ds(j * L, L)]
                ir = rbuf[_slot, s, pl.ds(HD + j * L, L)]
                rt = tbuf[_slot, s, pl.ds(j * L, L)]
                it = tbuf[_slot, s, pl.ds(HD + j * L, L)]
                acc = acc + rr * (rh * rt + ih * it) + ir * (rh * it - ih * rt)
            for sh in (8, 4, 2, 1):
                acc = acc + acc.at[lane ^ sh].get(mode="promise_in_bounds")
            pos = jnp.full((L,), _c * CH + s, dtype=jnp.int32)
            plsc.store_scatter(score_v, [pos], acc, mask=lane0)
            return carry

        lax.fori_loop(0, CH, body, 0)

    pltpu.sync_copy(score_v, out_hbm.at[pl.ds(wid * bw, bw)])


def kernel(sample, entity_embedding, relation_embedding):
    b = sample.shape[0]
    idx = sample.astype(jnp.int32)
    hi = idx[:, 0].reshape(NW, NCH, CH)
    ri = idx[:, 1].reshape(NW, NCH, CH)
    ti = idx[:, 2].reshape(NW, NCH, CH)

    mesh = plsc.VectorSubcoreMesh(core_axis_name="c", subcore_axis_name="s")
    run = functools.partial(
        pl.kernel,
        out_type=jax.ShapeDtypeStruct((b,), jnp.float32),
        mesh=mesh,
        compiler_params=pltpu.CompilerParams(needs_layout_passes=False),
        scratch_types=[
            pltpu.VMEM((NCH, CH), jnp.int32),
            pltpu.VMEM((NCH, CH), jnp.int32),
            pltpu.VMEM((NCH, CH), jnp.int32),
            pltpu.VMEM((2, CH, ED), jnp.float32),
            pltpu.VMEM((2, CH, ED), jnp.float32),
            pltpu.VMEM((2, CH, ED), jnp.float32),
            pltpu.VMEM((NCH * CH,), jnp.float32),
            pltpu.SemaphoreType.DMA,
            pltpu.SemaphoreType.DMA,
        ],
    )(_sc_body)
    score = run(hi, ri, ti, entity_embedding, relation_embedding)
    return score.reshape(b, 1)


# speedup vs baseline: 2.8475x; 1.0161x over previous
import functools

import jax
import jax.numpy as jnp
from jax import lax
from jax.experimental import pallas as pl
from jax.experimental.pallas import tpu as pltpu, tpu_sc as plsc

HD = 256
ED = 2 * HD
NW = 32
NCH = 16
CH = 32
L = 16


def _sc_body(hi_hbm, ri_hbm, ti_hbm, ent_hbm, rel_hbm, out_hbm,
             hi_v, ri_v, ti_v, hbuf, rbuf, tbuf, score_v, sem0, sem1):
    wid = lax.axis_index("s") * 2 + lax.axis_index("c")
    bw = NCH * CH

    pltpu.sync_copy(hi_hbm.at[wid], hi_v)
    pltpu.sync_copy(ri_hbm.at[wid], ri_v)
    pltpu.sync_copy(ti_hbm.at[wid], ti_v)

    sems = (sem0, sem1)

    def issue(c):
        slot = c & 1
        s = sems[slot]
        return (
            pltpu.async_copy(ent_hbm.at[hi_v.at[c]], hbuf.at[slot], s),
            pltpu.async_copy(rel_hbm.at[ri_v.at[c]], rbuf.at[slot], s),
            pltpu.async_copy(ent_hbm.at[ti_v.at[c]], tbuf.at[slot], s),
        )

    lane = lax.iota(jnp.int32, L)
    lane0 = lane == 0

    cps = [None, None]
    cps[0] = issue(0)
    for c in range(NCH):
        slot = c & 1
        if c + 1 < NCH:
            cps[(c + 1) & 1] = issue(c + 1)
        for cp in cps[slot]:
            cp.wait()

        @plsc.parallel_loop(0, CH)
        def body(s, _slot=slot, _c=c):
            acc = jnp.zeros((L,), jnp.float32)
            for j in range(HD // L):
                rh = hbuf[_slot, s, pl.ds(j * L, L)]
                ih = hbuf[_slot, s, pl.ds(HD + j * L, L)]
                rr = rbuf[_slot, s, pl.ds(j * L, L)]
                ir = rbuf[_slot, s, pl.ds(HD + j * L, L)]
                rt = tbuf[_slot, s, pl.ds(j * L, L)]
                it = tbuf[_slot, s, pl.ds(HD + j * L, L)]
                acc = acc + rr * (rh * rt + ih * it) + ir * (rh * it - ih * rt)
            for sh in (8, 4, 2, 1):
                acc = acc + acc.at[lane ^ sh].get(mode="promise_in_bounds")
            pos = jnp.full((L,), _c * CH + s, dtype=jnp.int32)
            plsc.store_scatter(score_v, [pos], acc, mask=lane0)

    pltpu.sync_copy(score_v, out_hbm.at[pl.ds(wid * bw, bw)])


def kernel(sample, entity_embedding, relation_embedding):
    b = sample.shape[0]
    idx = sample.astype(jnp.int32)
    hi = idx[:, 0].reshape(NW, NCH, CH)
    ri = idx[:, 1].reshape(NW, NCH, CH)
    ti = idx[:, 2].reshape(NW, NCH, CH)

    mesh = plsc.VectorSubcoreMesh(core_axis_name="c", subcore_axis_name="s")
    run = functools.partial(
        pl.kernel,
        out_type=jax.ShapeDtypeStruct((b,), jnp.float32),
        mesh=mesh,
        compiler_params=pltpu.CompilerParams(needs_layout_passes=False),
        scratch_types=[
            pltpu.VMEM((NCH, CH), jnp.int32),
            pltpu.VMEM((NCH, CH), jnp.int32),
            pltpu.VMEM((NCH, CH), jnp.int32),
            pltpu.VMEM((2, CH, ED), jnp.float32),
            pltpu.VMEM((2, CH, ED), jnp.float32),
            pltpu.VMEM((2, CH, ED), jnp.float32),
            pltpu.VMEM((NCH * CH,), jnp.float32),
            pltpu.SemaphoreType.DMA,
            pltpu.SemaphoreType.DMA,
        ],
    )(_sc_body)
    score = run(hi, ri, ti, entity_embedding, relation_embedding)
    return score.reshape(b, 1)
